# baseline (device time: 18649 ns/iter reference)
import os

import jax
import jax.numpy as jnp
from jax import lax
from jax.experimental import pallas as pl
from jax.experimental.pallas import tpu as pltpu

N_DEV = 32
Y_MASKS = (3, 4, 7)
Z_MASKS = (8, 16, 24)
CH = int(os.environ.get("KCHUNKS", "4"))


def kernel(x):
    _, m, n = x.shape
    half = m // 2
    rows = half // CH

    def body(
        x_ref,
        out_ref,
        acc_ref,
        comm0_ref,
        commy_ref,
        commz_ref,
        s0_send, s0_recv,
        sy_send, sy_recv,
        sz_send, sz_recv,
        s3_send, s3_recv,
        wave2_sem,
    ):
        my = lax.axis_index("i")
        xpeer = jnp.bitwise_xor(my, 1)
        ypeers = [jnp.bitwise_xor(my, mk) for mk in Y_MASKS]
        zpeers = [jnp.bitwise_xor(my, mk) for mk in Z_MASKS]
        oh = jnp.bitwise_and(jnp.bitwise_xor(my, my >> 1), 1)
        own0 = oh * half
        oth0 = half - own0

        barrier_sem = pltpu.get_barrier_semaphore()
        pl.semaphore_signal(
            barrier_sem,
            inc=1,
            device_id=(xpeer,),
            device_id_type=pl.DeviceIdType.MESH,
        )
        for peer in ypeers + zpeers:
            pl.semaphore_signal(
                wave2_sem,
                inc=1,
                device_id=(peer,),
                device_id_type=pl.DeviceIdType.MESH,
            )
        acc_ref[...] = x_ref[0, :, :].astype(jnp.bfloat16)
        pl.semaphore_wait(barrier_sem, 1)

        def exch(src_row, dst_ref, send_sem, recv_sem, peer):
            return pltpu.make_async_remote_copy(
                src_ref=acc_ref.at[pl.ds(src_row, rows)],
                dst_ref=dst_ref,
                send_sem=send_sem,
                recv_sem=recv_sem,
                device_id=(peer,),
                device_id_type=pl.DeviceIdType.MESH,
            )

        r0, r1, r2, r3 = {}, {}, {}, {}
        for c in range(CH):
            r0[c] = exch(
                oth0 + c * rows, comm0_ref.at[c], s0_send.at[c],
                s0_recv.at[c], xpeer,
            )
            r0[c].start()
        pl.semaphore_wait(wave2_sem, 6)

        def quad(c, use_y):
            if use_y:
                bufs, ss, sr, prs = commy_ref, sy_send, sy_recv, ypeers
            else:
                bufs, ss, sr, prs = commz_ref, sz_send, sz_recv, zpeers
            rs = [
                exch(
                    own0 + c * rows, bufs.at[k, c], ss.at[k, c],
                    sr.at[k, c], prs[k],
                )
                for k in range(3)
            ]
            add = lambda: (
                bufs[0, c, :, :] + bufs[1, c, :, :] + bufs[2, c, :, :]
            )
            return rs, add

        for c in range(CH):
            r0[c].wait()
            acc_ref[pl.ds(own0 + c * rows, rows), :] += comm0_ref[c, :, :]
            r1[c] = quad(c, use_y=(c % 2 == 0))
            for r in r1[c][0]:
                r.start()

        for c in range(CH):
            for r in r1[c][0]:
                r.wait()
            acc_ref[pl.ds(own0 + c * rows, rows), :] += r1[c][1]()
            r2[c] = quad(c, use_y=(c % 2 == 1))
            for r in r2[c][0]:
                r.start()

        for c in range(CH):
            for r in r2[c][0]:
                r.wait()
            acc_ref[pl.ds(own0 + c * rows, rows), :] += r2[c][1]()
            r3[c] = exch(
                own0 + c * rows,
                out_ref.at[pl.ds(own0 + c * rows, rows)],
                s3_send.at[c],
                s3_recv.at[c],
                xpeer,
            )
            r3[c].start()
            out_ref[pl.ds(own0 + c * rows, rows), :] = acc_ref[
                pl.ds(own0 + c * rows, rows), :
            ]

        for c in range(CH):
            r3[c].wait()

    return pl.pallas_call(
        body,
        out_shape=jax.ShapeDtypeStruct((m, n), jnp.bfloat16),
        in_specs=[pl.BlockSpec(memory_space=pltpu.VMEM)],
        out_specs=pl.BlockSpec(memory_space=pltpu.VMEM),
        scratch_shapes=[
            pltpu.VMEM((m, n), jnp.bfloat16),
            pltpu.VMEM((CH, rows, n), jnp.bfloat16),
            pltpu.VMEM((3, CH, rows, n), jnp.bfloat16),
            pltpu.VMEM((3, CH, rows, n), jnp.bfloat16),
            pltpu.SemaphoreType.DMA((CH,)),
            pltpu.SemaphoreType.DMA((CH,)),
            pltpu.SemaphoreType.DMA((3, CH)),
            pltpu.SemaphoreType.DMA((3, CH)),
            pltpu.SemaphoreType.DMA((3, CH)),
            pltpu.SemaphoreType.DMA((3, CH)),
            pltpu.SemaphoreType.DMA((CH,)),
            pltpu.SemaphoreType.DMA((CH,)),
            pltpu.SemaphoreType.REGULAR,
        ],
        compiler_params=pltpu.CompilerParams(collective_id=0),
    )(x)


# device time: 18633 ns/iter; 1.0009x vs baseline; 1.0009x over previous
import os

import jax
import jax.numpy as jnp
from jax import lax
from jax.experimental import pallas as pl
from jax.experimental.pallas import tpu as pltpu

N_DEV = 32
Y_MASKS = (3, 4, 7)
Z_MASKS = (8, 16, 24)
CH = int(os.environ.get("KCHUNKS", "4"))


def kernel(x):
    _, m, n = x.shape
    half = m // 2
    rows = half // CH

    def body(
        x_ref,
        out_ref,
        acc_ref,
        comm0_ref,
        commy_ref,
        commz_ref,
        comm3_ref,
        s0_send, s0_recv,
        sy_send, sy_recv,
        sz_send, sz_recv,
        s3_send, s3_recv,
        wave2_sem,
    ):
        my = lax.axis_index("i")
        xpeer = jnp.bitwise_xor(my, 1)
        ypeers = [jnp.bitwise_xor(my, mk) for mk in Y_MASKS]
        zpeers = [jnp.bitwise_xor(my, mk) for mk in Z_MASKS]
        oh = jnp.bitwise_and(jnp.bitwise_xor(my, my >> 1), 1)
        own0 = oh * half
        oth0 = half - own0

        barrier_sem = pltpu.get_barrier_semaphore()
        pl.semaphore_signal(
            barrier_sem,
            inc=1,
            device_id=(xpeer,),
            device_id_type=pl.DeviceIdType.MESH,
        )
        for peer in ypeers + zpeers:
            pl.semaphore_signal(
                wave2_sem,
                inc=1,
                device_id=(peer,),
                device_id_type=pl.DeviceIdType.MESH,
            )
        acc_ref[...] = x_ref[0, :, :].astype(jnp.bfloat16)
        pl.semaphore_wait(barrier_sem, 1)

        def exch(src_row, dst_ref, send_sem, recv_sem, peer):
            return pltpu.make_async_remote_copy(
                src_ref=acc_ref.at[pl.ds(src_row, rows)],
                dst_ref=dst_ref,
                send_sem=send_sem,
                recv_sem=recv_sem,
                device_id=(peer,),
                device_id_type=pl.DeviceIdType.MESH,
            )

        r0, r1, r2, r3 = {}, {}, {}, {}
        for c in range(CH):
            r0[c] = exch(
                oth0 + c * rows, comm0_ref.at[c], s0_send.at[c],
                s0_recv.at[c], xpeer,
            )
            r0[c].start()
        pl.semaphore_wait(wave2_sem, 6)

        def quad(c, use_y):
            if use_y:
                bufs, ss, sr, prs = commy_ref, sy_send, sy_recv, ypeers
            else:
                bufs, ss, sr, prs = commz_ref, sz_send, sz_recv, zpeers
            rs = [
                exch(
                    own0 + c * rows, bufs.at[k, c], ss.at[k, c],
                    sr.at[k, c], prs[k],
                )
                for k in range(3)
            ]
            add = lambda: (
                bufs[0, c, :, :] + bufs[1, c, :, :] + bufs[2, c, :, :]
            )
            return rs, add

        for c in range(CH):
            r0[c].wait()
            acc_ref[pl.ds(own0 + c * rows, rows), :] += comm0_ref[c, :, :]
            r1[c] = quad(c, use_y=(c % 2 == 0))
            for r in r1[c][0]:
                r.start()

        for c in range(CH):
            for r in r1[c][0]:
                r.wait()
            acc_ref[pl.ds(own0 + c * rows, rows), :] += r1[c][1]()
            r2[c] = quad(c, use_y=(c % 2 == 1))
            for r in r2[c][0]:
                r.start()

        for c in range(CH):
            for r in r2[c][0]:
                r.wait()
            acc_ref[pl.ds(own0 + c * rows, rows), :] += r2[c][1]()
            r3[c] = exch(
                own0 + c * rows, comm3_ref.at[c], s3_send.at[c],
                s3_recv.at[c], xpeer,
            )
            r3[c].start()
            out_ref[pl.ds(own0 + c * rows, rows), :] = acc_ref[
                pl.ds(own0 + c * rows, rows), :
            ]

        for c in range(CH):
            r3[c].wait()
            out_ref[pl.ds(oth0 + c * rows, rows), :] = comm3_ref[c, :, :]

    return pl.pallas_call(
        body,
        out_shape=jax.ShapeDtypeStruct((m, n), jnp.bfloat16),
        in_specs=[pl.BlockSpec(memory_space=pltpu.VMEM)],
        out_specs=pl.BlockSpec(memory_space=pltpu.VMEM),
        scratch_shapes=[
            pltpu.VMEM((m, n), jnp.bfloat16),
            pltpu.VMEM((CH, rows, n), jnp.bfloat16),
            pltpu.VMEM((3, CH, rows, n), jnp.bfloat16),
            pltpu.VMEM((3, CH, rows, n), jnp.bfloat16),
            pltpu.VMEM((CH, rows, n), jnp.bfloat16),
            pltpu.SemaphoreType.DMA((CH,)),
            pltpu.SemaphoreType.DMA((CH,)),
            pltpu.SemaphoreType.DMA((3, CH)),
            pltpu.SemaphoreType.DMA((3, CH)),
            pltpu.SemaphoreType.DMA((3, CH)),
            pltpu.SemaphoreType.DMA((3, CH)),
            pltpu.SemaphoreType.DMA((CH,)),
            pltpu.SemaphoreType.DMA((CH,)),
            pltpu.SemaphoreType.REGULAR,
        ],
        compiler_params=pltpu.CompilerParams(collective_id=0),
    )(x)


# device time: 17530 ns/iter; 1.0638x vs baseline; 1.0629x over previous
import os

import jax
import jax.numpy as jnp
from jax import lax
from jax.experimental import pallas as pl
from jax.experimental.pallas import tpu as pltpu

N_DEV = 32
Y_MASKS = (3, 4, 7)
Z_MASKS = (8, 16, 24)
CH = int(os.environ.get("KCHUNKS", "4"))


def kernel(x):
    _, m, n = x.shape
    half = m // 2
    rows = half // CH

    def body(
        x_ref,
        out_ref,
        acc_ref,
        comm0_ref,
        commy_ref,
        commz_ref,
        comm3_ref,
        s0_send, s0_recv,
        sy_send, sy_recv,
        sz_send, sz_recv,
        s3_send, s3_recv,
    ):
        my = lax.axis_index("i")
        xpeer = jnp.bitwise_xor(my, 1)
        ypeers = [jnp.bitwise_xor(my, mk) for mk in Y_MASKS]
        zpeers = [jnp.bitwise_xor(my, mk) for mk in Z_MASKS]
        oh = jnp.bitwise_and(jnp.bitwise_xor(my, my >> 1), 1)
        own0 = oh * half
        oth0 = half - own0

        barrier_sem = pltpu.get_barrier_semaphore()
        for peer in [xpeer] + ypeers + zpeers:
            pl.semaphore_signal(
                barrier_sem,
                inc=1,
                device_id=(peer,),
                device_id_type=pl.DeviceIdType.MESH,
            )
        acc_ref[...] = x_ref[0, :, :].astype(jnp.bfloat16)
        pl.semaphore_wait(barrier_sem, 7)

        def exch(src_row, dst_ref, send_sem, recv_sem, peer):
            return pltpu.make_async_remote_copy(
                src_ref=acc_ref.at[pl.ds(src_row, rows)],
                dst_ref=dst_ref,
                send_sem=send_sem,
                recv_sem=recv_sem,
                device_id=(peer,),
                device_id_type=pl.DeviceIdType.MESH,
            )

        r0, r1, r2, r3 = {}, {}, {}, {}
        for c in range(CH):
            r0[c] = exch(
                oth0 + c * rows, comm0_ref.at[c], s0_send.at[c],
                s0_recv.at[c], xpeer,
            )
            r0[c].start()

        def quad(c, use_y):
            if use_y:
                bufs, ss, sr, prs = commy_ref, sy_send, sy_recv, ypeers
            else:
                bufs, ss, sr, prs = commz_ref, sz_send, sz_recv, zpeers
            rs = [
                exch(
                    own0 + c * rows, bufs.at[k, c], ss.at[k, c],
                    sr.at[k, c], prs[k],
                )
                for k in range(3)
            ]
            add = lambda: (
                bufs[0, c, :, :] + bufs[1, c, :, :] + bufs[2, c, :, :]
            )
            return rs, add

        for c in range(CH):
            r0[c].wait()
            acc_ref[pl.ds(own0 + c * rows, rows), :] += comm0_ref[c, :, :]
            r1[c] = quad(c, use_y=(c % 2 == 0))
            for r in r1[c][0]:
                r.start()

        for c in range(CH):
            for r in r1[c][0]:
                r.wait()
            acc_ref[pl.ds(own0 + c * rows, rows), :] += r1[c][1]()
            r2[c] = quad(c, use_y=(c % 2 == 1))
            for r in r2[c][0]:
                r.start()

        for c in range(CH):
            for r in r2[c][0]:
                r.wait()
            acc_ref[pl.ds(own0 + c * rows, rows), :] += r2[c][1]()
            r3[c] = exch(
                own0 + c * rows, comm3_ref.at[c], s3_send.at[c],
                s3_recv.at[c], xpeer,
            )
            r3[c].start()
            out_ref[pl.ds(own0 + c * rows, rows), :] = acc_ref[
                pl.ds(own0 + c * rows, rows), :
            ]

        for c in range(CH):
            r3[c].wait()
            out_ref[pl.ds(oth0 + c * rows, rows), :] = comm3_ref[c, :, :]

    return pl.pallas_call(
        body,
        out_shape=jax.ShapeDtypeStruct((m, n), jnp.bfloat16),
        in_specs=[pl.BlockSpec(memory_space=pltpu.VMEM)],
        out_specs=pl.BlockSpec(memory_space=pltpu.VMEM),
        scratch_shapes=[
            pltpu.VMEM((m, n), jnp.bfloat16),
            pltpu.VMEM((CH, rows, n), jnp.bfloat16),
            pltpu.VMEM((3, CH, rows, n), jnp.bfloat16),
            pltpu.VMEM((3, CH, rows, n), jnp.bfloat16),
            pltpu.VMEM((CH, rows, n), jnp.bfloat16),
            pltpu.SemaphoreType.DMA((CH,)),
            pltpu.SemaphoreType.DMA((CH,)),
            pltpu.SemaphoreType.DMA((3, CH)),
            pltpu.SemaphoreType.DMA((3, CH)),
            pltpu.SemaphoreType.DMA((3, CH)),
            pltpu.SemaphoreType.DMA((3, CH)),
            pltpu.SemaphoreType.DMA((CH,)),
            pltpu.SemaphoreType.DMA((CH,)),
        ],
        compiler_params=pltpu.CompilerParams(collective_id=0),
    )(x)


# device time: 17527 ns/iter; 1.0640x vs baseline; 1.0002x over previous
import os

import jax
import jax.numpy as jnp
from jax import lax
from jax.experimental import pallas as pl
from jax.experimental.pallas import tpu as pltpu

N_DEV = 32
Y_MASKS = (3, 4, 7)
Z_MASKS = (8, 16, 24)
CH = int(os.environ.get("KCHUNKS", "8"))
EXCL = CH - 1
GT = EXCL + 2


def kernel(x):
    _, m, n = x.shape
    half = m // 2
    rows = half // CH

    def body(
        x_ref,
        out_ref,
        acc_ref,
        comm0_ref,
        commy_ref,
        commz_ref,
        comm3_ref,
        s0_send, s0_recv,
        sy_send, sy_recv,
        sz_send, sz_recv,
        s3_send, s3_recv,
    ):
        my = lax.axis_index("i")
        xpeer = jnp.bitwise_xor(my, 1)
        ypeers = [jnp.bitwise_xor(my, mk) for mk in Y_MASKS]
        zpeers = [jnp.bitwise_xor(my, mk) for mk in Z_MASKS]
        oh = jnp.bitwise_and(jnp.bitwise_xor(my, my >> 1), 1)
        own0 = oh * half
        oth0 = half - own0

        def row(g):
            if g < EXCL:
                return own0 + g * rows
            return (g - EXCL) * half + EXCL * rows

        barrier_sem = pltpu.get_barrier_semaphore()
        for peer in [xpeer] + ypeers + zpeers:
            pl.semaphore_signal(
                barrier_sem,
                inc=1,
                device_id=(peer,),
                device_id_type=pl.DeviceIdType.MESH,
            )
        acc_ref[...] = x_ref[0, :, :].astype(jnp.bfloat16)
        pl.semaphore_wait(barrier_sem, 7)

        def exch(src_row, dst_ref, send_sem, recv_sem, peer):
            return pltpu.make_async_remote_copy(
                src_ref=acc_ref.at[pl.ds(src_row, rows)],
                dst_ref=dst_ref,
                send_sem=send_sem,
                recv_sem=recv_sem,
                device_id=(peer,),
                device_id_type=pl.DeviceIdType.MESH,
            )

        r0, r1, r2, r3 = {}, {}, {}, {}
        for g in range(GT):
            src = oth0 + g * rows if g < EXCL else row(g)
            r0[g] = exch(
                src, comm0_ref.at[g], s0_send.at[g], s0_recv.at[g], xpeer,
            )
            r0[g].start()

        def quad(g, use_y):
            if use_y:
                bufs, ss, sr, prs = commy_ref, sy_send, sy_recv, ypeers
            else:
                bufs, ss, sr, prs = commz_ref, sz_send, sz_recv, zpeers
            rs = [
                exch(row(g), bufs.at[k, g], ss.at[k, g], sr.at[k, g], prs[k])
                for k in range(3)
            ]
            add = lambda: (
                bufs[0, g, :, :] + bufs[1, g, :, :] + bufs[2, g, :, :]
            )
            return rs, add

        for g in range(GT):
            r0[g].wait()
            acc_ref[pl.ds(row(g), rows), :] += comm0_ref[g, :, :]
            r1[g] = quad(g, use_y=(g % 2 == 0))
            for r in r1[g][0]:
                r.start()

        for g in range(GT):
            for r in r1[g][0]:
                r.wait()
            acc_ref[pl.ds(row(g), rows), :] += r1[g][1]()
            r2[g] = quad(g, use_y=(g % 2 == 1))
            for r in r2[g][0]:
                r.start()

        for g in range(GT):
            for r in r2[g][0]:
                r.wait()
            acc_ref[pl.ds(row(g), rows), :] += r2[g][1]()
            if g < EXCL:
                r3[g] = exch(
                    row(g), comm3_ref.at[g], s3_send.at[g], s3_recv.at[g],
                    xpeer,
                )
                r3[g].start()
            out_ref[pl.ds(row(g), rows), :] = acc_ref[pl.ds(row(g), rows), :]

        for g in range(EXCL):
            r3[g].wait()
            out_ref[pl.ds(oth0 + g * rows, rows), :] = comm3_ref[g, :, :]

    return pl.pallas_call(
        body,
        out_shape=jax.ShapeDtypeStruct((m, n), jnp.bfloat16),
        in_specs=[pl.BlockSpec(memory_space=pltpu.VMEM)],
        out_specs=pl.BlockSpec(memory_space=pltpu.VMEM),
        scratch_shapes=[
            pltpu.VMEM((m, n), jnp.bfloat16),
            pltpu.VMEM((GT, rows, n), jnp.bfloat16),
            pltpu.VMEM((3, GT, rows, n), jnp.bfloat16),
            pltpu.VMEM((3, GT, rows, n), jnp.bfloat16),
            pltpu.VMEM((EXCL, rows, n), jnp.bfloat16),
            pltpu.SemaphoreType.DMA((GT,)),
            pltpu.SemaphoreType.DMA((GT,)),
            pltpu.SemaphoreType.DMA((3, GT)),
            pltpu.SemaphoreType.DMA((3, GT)),
            pltpu.SemaphoreType.DMA((3, GT)),
            pltpu.SemaphoreType.DMA((3, GT)),
            pltpu.SemaphoreType.DMA((EXCL,)),
            pltpu.SemaphoreType.DMA((EXCL,)),
        ],
        compiler_params=pltpu.CompilerParams(collective_id=0),
    )(x)


# device time: 17468 ns/iter; 1.0676x vs baseline; 1.0034x over previous
import jax
import jax.numpy as jnp
from jax import lax
from jax.experimental import pallas as pl
from jax.experimental.pallas import tpu as pltpu

N_DEV = 32
Y_MASKS = (3, 4, 7)
Z_MASKS = (8, 16, 24)
CH = 8
EXCL = CH - 1
GT = EXCL + 2


def kernel(x):
    _, m, n = x.shape
    half = m // 2
    rows = half // CH

    def body(
        x_ref,
        out_ref,
        acc_ref,
        comm0_ref,
        commy_ref,
        commz_ref,
        comm3_ref,
        s0_send, s0_recv,
        sy_send, sy_recv,
        sz_send, sz_recv,
        s3_send, s3_recv,
    ):
        my = lax.axis_index("i")
        xpeer = jnp.bitwise_xor(my, 1)
        ypeers = [jnp.bitwise_xor(my, mk) for mk in Y_MASKS]
        zpeers = [jnp.bitwise_xor(my, mk) for mk in Z_MASKS]
        oh = jnp.bitwise_and(jnp.bitwise_xor(my, my >> 1), 1)
        own0 = oh * half
        oth0 = half - own0

        def row(g):
            if g < EXCL:
                return own0 + g * rows
            return (g - EXCL) * half + EXCL * rows

        barrier_sem = pltpu.get_barrier_semaphore()
        for peer in [xpeer] + ypeers + zpeers:
            pl.semaphore_signal(
                barrier_sem,
                inc=1,
                device_id=(peer,),
                device_id_type=pl.DeviceIdType.MESH,
            )
        acc_ref[...] = x_ref[0, :, :].astype(jnp.bfloat16)
        pl.semaphore_wait(barrier_sem, 7)

        def exch(src_row, dst_ref, send_sem, recv_sem, peer):
            return pltpu.make_async_remote_copy(
                src_ref=acc_ref.at[pl.ds(src_row, rows)],
                dst_ref=dst_ref,
                send_sem=send_sem,
                recv_sem=recv_sem,
                device_id=(peer,),
                device_id_type=pl.DeviceIdType.MESH,
            )

        r0, r1, r2, r3 = {}, {}, {}, {}
        for g in range(GT):
            src = oth0 + g * rows if g < EXCL else row(g)
            r0[g] = exch(
                src, comm0_ref.at[g], s0_send.at[g], s0_recv.at[g], xpeer,
            )
            r0[g].start()

        def quad(g, use_y):
            if use_y:
                bufs, ss, sr, prs = commy_ref, sy_send, sy_recv, ypeers
            else:
                bufs, ss, sr, prs = commz_ref, sz_send, sz_recv, zpeers
            rs = [
                exch(row(g), bufs.at[k, g], ss.at[k, g], sr.at[k, g], prs[k])
                for k in range(3)
            ]
            add = lambda: (
                bufs[0, g, :, :] + bufs[1, g, :, :] + bufs[2, g, :, :]
            )
            return rs, add

        for g in range(GT):
            r0[g].wait()
            acc_ref[pl.ds(row(g), rows), :] += comm0_ref[g, :, :]
            r1[g] = quad(g, use_y=(g % 2 == 0))
            for r in r1[g][0]:
                r.start()

        for g in range(GT):
            for r in r1[g][0]:
                r.wait()
            acc_ref[pl.ds(row(g), rows), :] += r1[g][1]()
            r2[g] = quad(g, use_y=(g % 2 == 1))
            for r in r2[g][0]:
                r.start()

        for g in range(GT):
            for r in r2[g][0]:
                r.wait()
            acc_ref[pl.ds(row(g), rows), :] += r2[g][1]()
            if g < EXCL:
                r3[g] = exch(
                    row(g), comm3_ref.at[g], s3_send.at[g], s3_recv.at[g],
                    xpeer,
                )
                r3[g].start()
            out_ref[pl.ds(row(g), rows), :] = acc_ref[pl.ds(row(g), rows), :]

        for g in range(EXCL):
            r3[g].wait()
            out_ref[pl.ds(oth0 + g * rows, rows), :] = comm3_ref[g, :, :]

    return pl.pallas_call(
        body,
        out_shape=jax.ShapeDtypeStruct((m, n), jnp.bfloat16),
        in_specs=[pl.BlockSpec(memory_space=pltpu.VMEM)],
        out_specs=pl.BlockSpec(memory_space=pltpu.VMEM),
        scratch_shapes=[
            pltpu.VMEM((m, n), jnp.bfloat16),
            pltpu.VMEM((GT, rows, n), jnp.bfloat16),
            pltpu.VMEM((3, GT, rows, n), jnp.bfloat16),
            pltpu.VMEM((3, GT, rows, n), jnp.bfloat16),
            pltpu.VMEM((EXCL, rows, n), jnp.bfloat16),
            pltpu.SemaphoreType.DMA((GT,)),
            pltpu.SemaphoreType.DMA((GT,)),
            pltpu.SemaphoreType.DMA((3, GT)),
            pltpu.SemaphoreType.DMA((3, GT)),
            pltpu.SemaphoreType.DMA((3, GT)),
            pltpu.SemaphoreType.DMA((3, GT)),
            pltpu.SemaphoreType.DMA((EXCL,)),
            pltpu.SemaphoreType.DMA((EXCL,)),
        ],
        compiler_params=pltpu.CompilerParams(collective_id=0),
    )(x)
